# baseline (speedup 1.0000x reference)
import dataclasses
import functools

import jax
import jax.numpy as jnp
from jax import lax
from jax.experimental import pallas as pl
from jax.experimental.pallas import tpu as pltpu
from jax.experimental.pallas import tpu_sc as plsc

N = 10000
E = 320000
D = 128
NC = 2
NS = 16
NW = NC * NS
EPW = E // NW
K = 128
PAD = 240
EPWP = EPW + PAD
B = EPWP // K
BC = 16
FB = EPW // K
TR = EPW - FB * K
NP = 10240
RPT = NP // NS

def _deg_body(src_hbm, dst_hbm, outa_hbm, outb_hbm,
              src_v, dst_v, ha_v, hb_v, sem):
    c = lax.axis_index("core")
    s = lax.axis_index("subcore")
    w = c * NS + s

    zv = jnp.zeros((16,), jnp.float32)
    ones = jnp.full((16,), 1.0, jnp.float32)

    @pl.loop(0, N // 16)
    def _(i):
        ha_v[pl.ds(i * 16, 16)] = zv
        hb_v[pl.ds(i * 16, 16)] = zv

    pltpu.async_copy(src_hbm.at[w], src_v, sem).wait()
    pltpu.async_copy(dst_hbm.at[w], dst_v, sem).wait()

    @pl.loop(0, FB)
    def _(b):
        @pl.loop(0, K // 16)
        def _(q):
            iva = src_v[b, pl.ds(q * 16, 16)]
            plsc.addupdate_scatter(ha_v, [iva], ones)
            ivb = dst_v[b, pl.ds(q * 16, 16)]
            plsc.addupdate_scatter(hb_v, [ivb], ones)

    @pl.loop(0, TR // 16)
    def _(q):
        iva = src_v[FB, pl.ds(q * 16, 16)]
        plsc.addupdate_scatter(ha_v, [iva], ones)
        ivb = dst_v[FB, pl.ds(q * 16, 16)]
        plsc.addupdate_scatter(hb_v, [ivb], ones)

    pltpu.async_copy(ha_v, outa_hbm.at[w], sem).wait()
    pltpu.async_copy(hb_v, outb_hbm.at[w], sem).wait()


def _agg_body(h_hbm, src_hbm, dst_hbm, out_hbm,
              src_v, dst_v, msg0_v, acc_sh, sem, sem0):
    c = lax.axis_index("core")
    s = lax.axis_index("subcore")
    w = c * NS + s

    zv = jnp.zeros((16,), jnp.float32)

    @pl.loop(0, K)
    def _(r):
        @pl.loop(0, D // 16)
        def _(q):
            msg0_v[r, pl.ds(q * 16, 16)] = zv

    @pl.loop(0, RPT // K)
    def _(j):
        pltpu.sync_copy(msg0_v, acc_sh.at[pl.ds(s * RPT + j * K, K)])

    pltpu.async_copy(src_hbm.at[w], src_v, sem).wait()
    pltpu.async_copy(dst_hbm.at[w], dst_v, sem).wait()
    plsc.subcore_barrier()

    @pl.loop(0, B)
    def _(b):
        pltpu.async_copy(h_hbm.at[src_v.at[b]], msg0_v, sem0).wait()
        pltpu.sync_copy(msg0_v, acc_sh.at[dst_v.at[b]], add=True)

    plsc.subcore_barrier()

    @pl.loop(0, RPT // K)
    def _(j):
        pltpu.sync_copy(acc_sh.at[pl.ds(s * RPT + j * K, K)], msg0_v)
        pltpu.sync_copy(msg0_v, out_hbm.at[c, pl.ds(s * RPT + j * K, K)])


_sc_kernels = {}


def _get_sc_kernels():
    if not _sc_kernels:
        mesh = plsc.VectorSubcoreMesh(core_axis_name="core",
                                      subcore_axis_name="subcore",
                                      num_cores=NC, num_subcores=NS)
        cp = pltpu.CompilerParams()
        if "needs_layout_passes" in pltpu.CompilerParams.__dataclass_fields__:
            cp = dataclasses.replace(cp, needs_layout_passes=False)
        _sc_kernels["deg"] = pl.kernel(
            _deg_body,
            out_type=(jax.ShapeDtypeStruct((NW, N), jnp.float32),
                      jax.ShapeDtypeStruct((NW, N), jnp.float32)),
            mesh=mesh,
            compiler_params=cp,
            scratch_types=[
                pltpu.VMEM((B, K), jnp.int32),
                pltpu.VMEM((B, K), jnp.int32),
                pltpu.VMEM((N,), jnp.float32),
                pltpu.VMEM((N,), jnp.float32),
                pltpu.SemaphoreType.DMA,
            ],
        )
        _sc_kernels["agg"] = pl.kernel(
            _agg_body,
            out_type=jax.ShapeDtypeStruct((NC, NP, D), jnp.float32),
            mesh=mesh,
            scratch_types=[
                pltpu.VMEM((B, K), jnp.int32),
                pltpu.VMEM((B, K), jnp.int32),
                pltpu.VMEM((K, D), jnp.float32),
                pltpu.VMEM_SHARED((NP, D), jnp.float32),
                pltpu.SemaphoreType.DMA,
                pltpu.SemaphoreType.DMA,
            ],
        )
    return _sc_kernels


_RB = 1000


def _norms(dega_blk, degb_blk):
    do = jnp.sum(dega_blk, axis=1)[:, None]
    di = jnp.sum(degb_blk, axis=1)[:, None]
    ns = lax.rsqrt(jnp.maximum(do, 1.0))
    nd = lax.rsqrt(jnp.maximum(di, 1.0))
    return ns, nd


def _t_first_body(dega_ref, degb_ref, x_ref, w_ref, out_ref):
    ns, _ = _norms(dega_ref[...], degb_ref[...])
    out_ref[...] = jnp.dot(x_ref[...] * ns, w_ref[...],
                           preferred_element_type=jnp.float32)


def _t_first(dega, degb, x, w):
    return pl.pallas_call(
        _t_first_body,
        grid=(N // _RB,),
        in_specs=[
            pl.BlockSpec((_RB, NW), lambda i: (i, 0)),
            pl.BlockSpec((_RB, NW), lambda i: (i, 0)),
            pl.BlockSpec((_RB, D), lambda i: (i, 0)),
            pl.BlockSpec((D, D), lambda i: (0, 0)),
        ],
        out_specs=pl.BlockSpec((_RB, D), lambda i: (i, 0)),
        out_shape=jax.ShapeDtypeStruct((N, D), jnp.float32),
    )(dega, degb, x, w)


def _t_mid_body(dega_ref, degb_ref, p_ref, b_ref, w_ref, out_ref):
    ns, nd = _norms(dega_ref[...], degb_ref[...])
    agg = p_ref[0] + p_ref[1]
    x = jnp.maximum(agg * nd + b_ref[...], 0.0)
    out_ref[...] = jnp.dot(x * ns, w_ref[...],
                           preferred_element_type=jnp.float32)


def _t_mid(dega, degb, p, bias, w):
    return pl.pallas_call(
        _t_mid_body,
        grid=(N // _RB,),
        in_specs=[
            pl.BlockSpec((_RB, NW), lambda i: (i, 0)),
            pl.BlockSpec((_RB, NW), lambda i: (i, 0)),
            pl.BlockSpec((NC, _RB, D), lambda i: (0, i, 0)),
            pl.BlockSpec((1, D), lambda i: (0, 0)),
            pl.BlockSpec((D, D), lambda i: (0, 0)),
        ],
        out_specs=pl.BlockSpec((_RB, D), lambda i: (i, 0)),
        out_shape=jax.ShapeDtypeStruct((N, D), jnp.float32),
    )(dega, degb, p, bias, w)


def _t_final_body(dega_ref, degb_ref, p_ref, b_ref, wp1_ref, bp1_ref,
                  wp2_ref, bp2_ref, out_ref):
    _, nd = _norms(dega_ref[...], degb_ref[...])
    agg = p_ref[0] + p_ref[1]
    x = jnp.maximum(agg * nd + b_ref[...], 0.0)
    m = jnp.mean(x, axis=0, keepdims=True)
    hid = jnp.maximum(jnp.dot(m, wp1_ref[...],
                              preferred_element_type=jnp.float32)
                      + bp1_ref[...], 0.0)
    out_ref[...] = jnp.dot(hid, wp2_ref[...],
                           preferred_element_type=jnp.float32) + bp2_ref[...]


def _t_final(dega, degb, p, bias, wp1, bp1, wp2, bp2):
    n_class = wp2.shape[1]
    mid = wp1.shape[1]
    return pl.pallas_call(
        _t_final_body,
        grid=(1,),
        in_specs=[
            pl.BlockSpec((N, NW), lambda i: (0, 0)),
            pl.BlockSpec((N, NW), lambda i: (0, 0)),
            pl.BlockSpec((NC, N, D), lambda i: (0, 0, 0)),
            pl.BlockSpec((1, D), lambda i: (0, 0)),
            pl.BlockSpec((D, mid), lambda i: (0, 0)),
            pl.BlockSpec((1, mid), lambda i: (0, 0)),
            pl.BlockSpec((mid, n_class), lambda i: (0, 0)),
            pl.BlockSpec((1, n_class), lambda i: (0, 0)),
        ],
        out_specs=pl.BlockSpec((1, n_class), lambda i: (0, 0)),
        out_shape=jax.ShapeDtypeStruct((1, n_class), jnp.float32),
    )(dega, degb, p, bias, wp1, bp1, wp2, bp2)


def kernel(feat, edge_index, W1, b1, W2, b2, W3, b3, W4, b4, W5, b5, W6, b6,
           Wp1, bp1, Wp2, bp2):
    src = jnp.pad(edge_index[0].reshape(NW, EPW),
                  ((0, 0), (0, PAD))).reshape(NW, B, K)
    dst = jnp.pad(edge_index[1].reshape(NW, EPW), ((0, 0), (0, PAD)),
                  constant_values=N).reshape(NW, B, K)
    sck = _get_sc_kernels()
    dega, degb = sck["deg"](src, dst)
    dega = dega.T
    degb = degb.T

    h = _t_first(dega, degb, feat, W1)
    mids = [(b1, W2), (b2, W3), (b3, W4), (b4, W5), (b5, W6)]
    for bias, w in mids:
        p = sck["agg"](h, src, dst)
        h = _t_mid(dega, degb, p, bias.reshape(1, D), w)
    p = sck["agg"](h, src, dst)

    return _t_final(dega, degb, p, b6.reshape(1, D), Wp1, bp1.reshape(1, -1),
                    Wp2, bp2.reshape(1, -1))


# speedup vs baseline: 1.0869x; 1.0869x over previous
import dataclasses
import functools

import jax
import jax.numpy as jnp
from jax import lax
from jax.experimental import pallas as pl
from jax.experimental.pallas import tpu as pltpu
from jax.experimental.pallas import tpu_sc as plsc

N = 10000
E = 320000
D = 128
NC = 2
NS = 16
NW = NC * NS
EPW = E // NW
K = 80
PAD = 240
EPWP = EPW + PAD
B = EPWP // K
BC = 16
FB = EPW // K
NP = 10240
RPT = NP // NS

def _deg_body(src_hbm, dst_hbm, outa_hbm, outb_hbm,
              src_v, dst_v, ha_v, hb_v, sem):
    c = lax.axis_index("core")
    s = lax.axis_index("subcore")
    w = c * NS + s

    zv = jnp.zeros((16,), jnp.float32)
    ones = jnp.full((16,), 1.0, jnp.float32)

    @pl.loop(0, N // 16)
    def _(i):
        ha_v[pl.ds(i * 16, 16)] = zv
        hb_v[pl.ds(i * 16, 16)] = zv

    pltpu.async_copy(src_hbm.at[w], src_v, sem).wait()
    pltpu.async_copy(dst_hbm.at[w], dst_v, sem).wait()

    @pl.loop(0, FB)
    def _(b):
        @pl.loop(0, K // 16)
        def _(q):
            iva = src_v[b, pl.ds(q * 16, 16)]
            plsc.addupdate_scatter(ha_v, [iva], ones)
            ivb = dst_v[b, pl.ds(q * 16, 16)]
            plsc.addupdate_scatter(hb_v, [ivb], ones)

    pltpu.async_copy(ha_v, outa_hbm.at[w], sem).wait()
    pltpu.async_copy(hb_v, outb_hbm.at[w], sem).wait()


def _agg_body(h_hbm, src_hbm, dst_hbm, out_hbm,
              src_v, dst_v, msg0_v, msg1_v, acc_sh, sem, sem0, sem1):
    c = lax.axis_index("core")
    s = lax.axis_index("subcore")
    w = c * NS + s

    zv = jnp.zeros((16,), jnp.float32)

    @pl.loop(0, K)
    def _(r):
        @pl.loop(0, D // 16)
        def _(q):
            msg0_v[r, pl.ds(q * 16, 16)] = zv

    @pl.loop(0, RPT // K)
    def _(j):
        pltpu.sync_copy(msg0_v, acc_sh.at[pl.ds(s * RPT + j * K, K)])

    plsc.subcore_barrier()

    @pl.loop(0, B // BC)
    def _(ci):
        pltpu.async_copy(src_hbm.at[w, pl.ds(ci * BC, BC)], src_v, sem).wait()
        pltpu.async_copy(dst_hbm.at[w, pl.ds(ci * BC, BC)], dst_v, sem).wait()
        pltpu.async_copy(h_hbm.at[src_v.at[0]], msg0_v, sem0)

        @pl.loop(0, BC // 2 - 1)
        def _(i):
            b0 = i * 2
            pltpu.async_copy(h_hbm.at[src_v.at[b0 + 1]], msg1_v, sem1)
            pltpu.make_async_copy(h_hbm.at[src_v.at[b0]], msg0_v, sem0).wait()
            pltpu.sync_copy(msg0_v, acc_sh.at[dst_v.at[b0]], add=True)
            pltpu.async_copy(h_hbm.at[src_v.at[b0 + 2]], msg0_v, sem0)
            pltpu.make_async_copy(h_hbm.at[src_v.at[b0 + 1]], msg1_v,
                                  sem1).wait()
            pltpu.sync_copy(msg1_v, acc_sh.at[dst_v.at[b0 + 1]], add=True)

        pltpu.async_copy(h_hbm.at[src_v.at[BC - 1]], msg1_v, sem1)
        pltpu.make_async_copy(h_hbm.at[src_v.at[BC - 2]], msg0_v, sem0).wait()
        pltpu.sync_copy(msg0_v, acc_sh.at[dst_v.at[BC - 2]], add=True)
        pltpu.make_async_copy(h_hbm.at[src_v.at[BC - 1]], msg1_v, sem1).wait()
        pltpu.sync_copy(msg1_v, acc_sh.at[dst_v.at[BC - 1]], add=True)

    plsc.subcore_barrier()

    @pl.loop(0, RPT // K)
    def _(j):
        pltpu.sync_copy(acc_sh.at[pl.ds(s * RPT + j * K, K)], msg0_v)
        pltpu.sync_copy(msg0_v, out_hbm.at[c, pl.ds(s * RPT + j * K, K)])


_sc_kernels = {}


def _get_sc_kernels():
    if not _sc_kernels:
        mesh = plsc.VectorSubcoreMesh(core_axis_name="core",
                                      subcore_axis_name="subcore",
                                      num_cores=NC, num_subcores=NS)
        cp = pltpu.CompilerParams()
        if "needs_layout_passes" in pltpu.CompilerParams.__dataclass_fields__:
            cp = dataclasses.replace(cp, needs_layout_passes=False)
        _sc_kernels["deg"] = pl.kernel(
            _deg_body,
            out_type=(jax.ShapeDtypeStruct((NW, N), jnp.float32),
                      jax.ShapeDtypeStruct((NW, N), jnp.float32)),
            mesh=mesh,
            compiler_params=cp,
            scratch_types=[
                pltpu.VMEM((B, K), jnp.int32),
                pltpu.VMEM((B, K), jnp.int32),
                pltpu.VMEM((N,), jnp.float32),
                pltpu.VMEM((N,), jnp.float32),
                pltpu.SemaphoreType.DMA,
            ],
        )
        _sc_kernels["agg"] = pl.kernel(
            _agg_body,
            out_type=jax.ShapeDtypeStruct((NC, NP, D), jnp.float32),
            mesh=mesh,
            scratch_types=[
                pltpu.VMEM((BC, K), jnp.int32),
                pltpu.VMEM((BC, K), jnp.int32),
                pltpu.VMEM((K, D), jnp.float32),
                pltpu.VMEM((K, D), jnp.float32),
                pltpu.VMEM_SHARED((NP, D), jnp.float32),
                pltpu.SemaphoreType.DMA,
                pltpu.SemaphoreType.DMA,
                pltpu.SemaphoreType.DMA,
            ],
        )
    return _sc_kernels


_RB = 1000


def _norms(dega_blk, degb_blk):
    do = jnp.sum(dega_blk, axis=1)[:, None]
    di = jnp.sum(degb_blk, axis=1)[:, None]
    ns = lax.rsqrt(jnp.maximum(do, 1.0))
    nd = lax.rsqrt(jnp.maximum(di, 1.0))
    return ns, nd


def _t_first_body(dega_ref, degb_ref, x_ref, w_ref, out_ref):
    ns, _ = _norms(dega_ref[...], degb_ref[...])
    out_ref[...] = jnp.dot(x_ref[...] * ns, w_ref[...],
                           preferred_element_type=jnp.float32)


def _t_first(dega, degb, x, w):
    return pl.pallas_call(
        _t_first_body,
        grid=(N // _RB,),
        in_specs=[
            pl.BlockSpec((_RB, NW), lambda i: (i, 0)),
            pl.BlockSpec((_RB, NW), lambda i: (i, 0)),
            pl.BlockSpec((_RB, D), lambda i: (i, 0)),
            pl.BlockSpec((D, D), lambda i: (0, 0)),
        ],
        out_specs=pl.BlockSpec((_RB, D), lambda i: (i, 0)),
        out_shape=jax.ShapeDtypeStruct((N, D), jnp.float32),
    )(dega, degb, x, w)


def _t_mid_body(dega_ref, degb_ref, p_ref, b_ref, w_ref, out_ref):
    ns, nd = _norms(dega_ref[...], degb_ref[...])
    agg = p_ref[0] + p_ref[1]
    x = jnp.maximum(agg * nd + b_ref[...], 0.0)
    out_ref[...] = jnp.dot(x * ns, w_ref[...],
                           preferred_element_type=jnp.float32)


def _t_mid(dega, degb, p, bias, w):
    return pl.pallas_call(
        _t_mid_body,
        grid=(N // _RB,),
        in_specs=[
            pl.BlockSpec((_RB, NW), lambda i: (i, 0)),
            pl.BlockSpec((_RB, NW), lambda i: (i, 0)),
            pl.BlockSpec((NC, _RB, D), lambda i: (0, i, 0)),
            pl.BlockSpec((1, D), lambda i: (0, 0)),
            pl.BlockSpec((D, D), lambda i: (0, 0)),
        ],
        out_specs=pl.BlockSpec((_RB, D), lambda i: (i, 0)),
        out_shape=jax.ShapeDtypeStruct((N, D), jnp.float32),
    )(dega, degb, p, bias, w)


def _t_final_body(dega_ref, degb_ref, p_ref, b_ref, wp1_ref, bp1_ref,
                  wp2_ref, bp2_ref, out_ref):
    _, nd = _norms(dega_ref[...], degb_ref[...])
    agg = p_ref[0] + p_ref[1]
    x = jnp.maximum(agg * nd + b_ref[...], 0.0)
    m = jnp.mean(x, axis=0, keepdims=True)
    hid = jnp.maximum(jnp.dot(m, wp1_ref[...],
                              preferred_element_type=jnp.float32)
                      + bp1_ref[...], 0.0)
    out_ref[...] = jnp.dot(hid, wp2_ref[...],
                           preferred_element_type=jnp.float32) + bp2_ref[...]


def _t_final(dega, degb, p, bias, wp1, bp1, wp2, bp2):
    n_class = wp2.shape[1]
    mid = wp1.shape[1]
    return pl.pallas_call(
        _t_final_body,
        grid=(1,),
        in_specs=[
            pl.BlockSpec((N, NW), lambda i: (0, 0)),
            pl.BlockSpec((N, NW), lambda i: (0, 0)),
            pl.BlockSpec((NC, N, D), lambda i: (0, 0, 0)),
            pl.BlockSpec((1, D), lambda i: (0, 0)),
            pl.BlockSpec((D, mid), lambda i: (0, 0)),
            pl.BlockSpec((1, mid), lambda i: (0, 0)),
            pl.BlockSpec((mid, n_class), lambda i: (0, 0)),
            pl.BlockSpec((1, n_class), lambda i: (0, 0)),
        ],
        out_specs=pl.BlockSpec((1, n_class), lambda i: (0, 0)),
        out_shape=jax.ShapeDtypeStruct((1, n_class), jnp.float32),
    )(dega, degb, p, bias, wp1, bp1, wp2, bp2)


def kernel(feat, edge_index, W1, b1, W2, b2, W3, b3, W4, b4, W5, b5, W6, b6,
           Wp1, bp1, Wp2, bp2):
    src = jnp.pad(edge_index[0].reshape(NW, EPW),
                  ((0, 0), (0, PAD))).reshape(NW, B, K)
    dst = jnp.pad(edge_index[1].reshape(NW, EPW), ((0, 0), (0, PAD)),
                  constant_values=N).reshape(NW, B, K)
    sck = _get_sc_kernels()
    dega, degb = sck["deg"](src, dst)
    dega = dega.T
    degb = degb.T

    h = _t_first(dega, degb, feat, W1)
    mids = [(b1, W2), (b2, W3), (b3, W4), (b4, W5), (b5, W6)]
    for bias, w in mids:
        p = sck["agg"](h, src, dst)
        h = _t_mid(dega, degb, p, bias.reshape(1, D), w)
    p = sck["agg"](h, src, dst)

    return _t_final(dega, degb, p, b6.reshape(1, D), Wp1, bp1.reshape(1, -1),
                    Wp2, bp2.reshape(1, -1))


# speedup vs baseline: 3.0264x; 2.7845x over previous
import dataclasses
import functools

import jax
import jax.numpy as jnp
from jax import lax
from jax.experimental import pallas as pl
from jax.experimental.pallas import tpu as pltpu
from jax.experimental.pallas import tpu_sc as plsc

N = 10000
E = 320000
D = 128
NC = 2
NS = 16
NW = NC * NS
EPW = E // NW
K = 80
PAD = 240
EPWP = EPW + PAD
B = EPWP // K
BC = 16
FB = EPW // K
NP = 10240
RPT = NP // NS

def _deg_body(src_hbm, dst_hbm, outa_hbm, outb_hbm,
              src_v, dst_v, ha_v, hb_v, sem):
    c = lax.axis_index("core")
    s = lax.axis_index("subcore")
    w = c * NS + s

    zv = jnp.zeros((16,), jnp.float32)
    ones = jnp.full((16,), 1.0, jnp.float32)

    @pl.loop(0, N // 16)
    def _(i):
        ha_v[pl.ds(i * 16, 16)] = zv
        hb_v[pl.ds(i * 16, 16)] = zv

    pltpu.async_copy(src_hbm.at[w], src_v, sem).wait()
    pltpu.async_copy(dst_hbm.at[w], dst_v, sem).wait()

    @pl.loop(0, FB)
    def _(b):
        @pl.loop(0, K // 16)
        def _(q):
            iva = src_v[b, pl.ds(q * 16, 16)]
            plsc.addupdate_scatter(ha_v, [iva], ones)
            ivb = dst_v[b, pl.ds(q * 16, 16)]
            plsc.addupdate_scatter(hb_v, [ivb], ones)

    pltpu.async_copy(ha_v, outa_hbm.at[w], sem).wait()
    pltpu.async_copy(hb_v, outb_hbm.at[w], sem).wait()


def _agg_body(h_hbm, src_hbm, dst_hbm, out_hbm,
              src_v, dst_v, msg0_v, msg1_v, acc_sh, sem, sem0, sem1):
    c = lax.axis_index("core")
    s = lax.axis_index("subcore")
    w = c * NS + s

    zv = jnp.zeros((16,), jnp.float32)

    @pl.loop(0, K)
    def _(r):
        @pl.loop(0, D // 16)
        def _(q):
            msg0_v[r, pl.ds(q * 16, 16)] = zv

    @pl.loop(0, RPT // K)
    def _(j):
        pltpu.sync_copy(msg0_v, acc_sh.at[pl.ds(s * RPT + j * K, K)])

    plsc.subcore_barrier()

    @pl.loop(0, B // BC)
    def _(ci):
        pltpu.async_copy(src_hbm.at[w, pl.ds(ci * BC, BC)], src_v, sem).wait()
        pltpu.async_copy(dst_hbm.at[w, pl.ds(ci * BC, BC)], dst_v, sem).wait()
        pltpu.async_copy(h_hbm.at[src_v.at[0]], msg0_v, sem0)

        @pl.loop(0, BC // 2 - 1)
        def _(i):
            b0 = i * 2
            pltpu.async_copy(h_hbm.at[src_v.at[b0 + 1]], msg1_v, sem1)
            pltpu.make_async_copy(h_hbm.at[src_v.at[b0]], msg0_v, sem0).wait()
            pltpu.sync_copy(msg0_v, acc_sh.at[dst_v.at[b0]], add=True)
            pltpu.async_copy(h_hbm.at[src_v.at[b0 + 2]], msg0_v, sem0)
            pltpu.make_async_copy(h_hbm.at[src_v.at[b0 + 1]], msg1_v,
                                  sem1).wait()
            pltpu.sync_copy(msg1_v, acc_sh.at[dst_v.at[b0 + 1]], add=True)

        pltpu.async_copy(h_hbm.at[src_v.at[BC - 1]], msg1_v, sem1)
        pltpu.make_async_copy(h_hbm.at[src_v.at[BC - 2]], msg0_v, sem0).wait()
        pltpu.sync_copy(msg0_v, acc_sh.at[dst_v.at[BC - 2]], add=True)
        pltpu.make_async_copy(h_hbm.at[src_v.at[BC - 1]], msg1_v, sem1).wait()
        pltpu.sync_copy(msg1_v, acc_sh.at[dst_v.at[BC - 1]], add=True)

    plsc.subcore_barrier()

    @pl.loop(0, RPT // K)
    def _(j):
        pltpu.sync_copy(acc_sh.at[pl.ds(s * RPT + j * K, K)], msg0_v)
        pltpu.sync_copy(msg0_v, out_hbm.at[c, pl.ds(s * RPT + j * K, K)])


_sc_kernels = {}


def _get_sc_kernels():
    if not _sc_kernels:
        mesh = plsc.VectorSubcoreMesh(core_axis_name="core",
                                      subcore_axis_name="subcore",
                                      num_cores=NC, num_subcores=NS)
        cp = pltpu.CompilerParams()
        if "needs_layout_passes" in pltpu.CompilerParams.__dataclass_fields__:
            cp = dataclasses.replace(cp, needs_layout_passes=False)
        _sc_kernels["deg"] = pl.kernel(
            _deg_body,
            out_type=(jax.ShapeDtypeStruct((NW, N), jnp.float32),
                      jax.ShapeDtypeStruct((NW, N), jnp.float32)),
            mesh=mesh,
            compiler_params=cp,
            scratch_types=[
                pltpu.VMEM((B, K), jnp.int32),
                pltpu.VMEM((B, K), jnp.int32),
                pltpu.VMEM((N,), jnp.float32),
                pltpu.VMEM((N,), jnp.float32),
                pltpu.SemaphoreType.DMA,
            ],
        )
        _sc_kernels["agg"] = pl.kernel(
            _agg_body,
            out_type=jax.ShapeDtypeStruct((NC, NP, D), jnp.float32),
            mesh=mesh,
            scratch_types=[
                pltpu.VMEM((BC, K), jnp.int32),
                pltpu.VMEM((BC, K), jnp.int32),
                pltpu.VMEM((K, D), jnp.float32),
                pltpu.VMEM((K, D), jnp.float32),
                pltpu.VMEM_SHARED((NP, D), jnp.float32),
                pltpu.SemaphoreType.DMA,
                pltpu.SemaphoreType.DMA,
                pltpu.SemaphoreType.DMA,
            ],
        )
    return _sc_kernels


_RB = 1000


def _norms(dega_blk, degb_blk):
    do = jnp.sum(dega_blk, axis=1)[:, None]
    di = jnp.sum(degb_blk, axis=1)[:, None]
    ns = lax.rsqrt(jnp.maximum(do, 1.0))
    nd = lax.rsqrt(jnp.maximum(di, 1.0))
    return ns, nd


def _t_first_body(dega_ref, degb_ref, x_ref, w_ref, out_ref):
    ns, _ = _norms(dega_ref[...], degb_ref[...])
    out_ref[...] = jnp.dot(x_ref[...] * ns, w_ref[...],
                           preferred_element_type=jnp.float32)


def _t_first(dega, degb, x, w):
    return pl.pallas_call(
        _t_first_body,
        grid=(N // _RB,),
        in_specs=[
            pl.BlockSpec((_RB, NW), lambda i: (i, 0)),
            pl.BlockSpec((_RB, NW), lambda i: (i, 0)),
            pl.BlockSpec((_RB, D), lambda i: (i, 0)),
            pl.BlockSpec((D, D), lambda i: (0, 0)),
        ],
        out_specs=pl.BlockSpec((_RB, D), lambda i: (i, 0)),
        out_shape=jax.ShapeDtypeStruct((N, D), jnp.float32),
    )(dega, degb, x, w)


def _t_mid_body(dega_ref, degb_ref, p_ref, b_ref, w_ref, out_ref):
    ns, nd = _norms(dega_ref[...], degb_ref[...])
    agg = p_ref[0] + p_ref[1]
    x = jnp.maximum(agg * nd + b_ref[...], 0.0)
    out_ref[...] = jnp.dot(x * ns, w_ref[...],
                           preferred_element_type=jnp.float32)


def _t_mid(dega, degb, p, bias, w):
    return pl.pallas_call(
        _t_mid_body,
        grid=(N // _RB,),
        in_specs=[
            pl.BlockSpec((_RB, NW), lambda i: (i, 0)),
            pl.BlockSpec((_RB, NW), lambda i: (i, 0)),
            pl.BlockSpec((NC, _RB, D), lambda i: (0, i, 0)),
            pl.BlockSpec((1, D), lambda i: (0, 0)),
            pl.BlockSpec((D, D), lambda i: (0, 0)),
        ],
        out_specs=pl.BlockSpec((_RB, D), lambda i: (i, 0)),
        out_shape=jax.ShapeDtypeStruct((N, D), jnp.float32),
    )(dega, degb, p, bias, w)


def _t_final_body(dega_ref, degb_ref, p_ref, b_ref, wp1_ref, bp1_ref,
                  wp2_ref, bp2_ref, out_ref):
    _, nd = _norms(dega_ref[...], degb_ref[...])
    agg = p_ref[0] + p_ref[1]
    x = jnp.maximum(agg * nd + b_ref[...], 0.0)
    m = jnp.mean(x, axis=0, keepdims=True)
    hid = jnp.maximum(jnp.dot(m, wp1_ref[...],
                              preferred_element_type=jnp.float32)
                      + bp1_ref[...], 0.0)
    out_ref[...] = jnp.dot(hid, wp2_ref[...],
                           preferred_element_type=jnp.float32) + bp2_ref[...]


def _t_final(dega, degb, p, bias, wp1, bp1, wp2, bp2):
    n_class = wp2.shape[1]
    mid = wp1.shape[1]
    return pl.pallas_call(
        _t_final_body,
        grid=(1,),
        in_specs=[
            pl.BlockSpec((N, NW), lambda i: (0, 0)),
            pl.BlockSpec((N, NW), lambda i: (0, 0)),
            pl.BlockSpec((NC, N, D), lambda i: (0, 0, 0)),
            pl.BlockSpec((1, D), lambda i: (0, 0)),
            pl.BlockSpec((D, mid), lambda i: (0, 0)),
            pl.BlockSpec((1, mid), lambda i: (0, 0)),
            pl.BlockSpec((mid, n_class), lambda i: (0, 0)),
            pl.BlockSpec((1, n_class), lambda i: (0, 0)),
        ],
        out_specs=pl.BlockSpec((1, n_class), lambda i: (0, 0)),
        out_shape=jax.ShapeDtypeStruct((1, n_class), jnp.float32),
    )(dega, degb, p, bias, wp1, bp1, wp2, bp2)


def kernel(feat, edge_index, W1, b1, W2, b2, W3, b3, W4, b4, W5, b5, W6, b6,
           Wp1, bp1, Wp2, bp2):
    src_pad = jnp.broadcast_to(jnp.arange(PAD, dtype=jnp.int32), (NW, PAD))
    dst_pad = jnp.broadcast_to(N + jnp.arange(PAD, dtype=jnp.int32), (NW, PAD))
    src = jnp.concatenate([edge_index[0].reshape(NW, EPW), src_pad],
                          axis=1).reshape(NW, B, K)
    dst = jnp.concatenate([edge_index[1].reshape(NW, EPW), dst_pad],
                          axis=1).reshape(NW, B, K)
    sck = _get_sc_kernels()
    dega, degb = sck["deg"](src, dst)
    dega = dega.T
    degb = degb.T

    h = _t_first(dega, degb, feat, W1)
    mids = [(b1, W2), (b2, W3), (b3, W4), (b4, W5), (b5, W6)]
    for bias, w in mids:
        p = sck["agg"](h, src, dst)
        h = _t_mid(dega, degb, p, bias.reshape(1, D), w)
    p = sck["agg"](h, src, dst)

    return _t_final(dega, degb, p, b6.reshape(1, D), Wp1, bp1.reshape(1, -1),
                    Wp2, bp2.reshape(1, -1))


# speedup vs baseline: 3.1955x; 1.0559x over previous
import dataclasses
import functools

import jax
import jax.numpy as jnp
from jax import lax
from jax.experimental import pallas as pl
from jax.experimental.pallas import tpu as pltpu
from jax.experimental.pallas import tpu_sc as plsc

N = 10000
E = 320000
D = 128
NC = 2
NS = 16
NW = NC * NS
EPW = E // NW
K = 80
PAD = 240
EPWP = EPW + PAD
B = EPWP // K
BC = 32
FB = EPW // K
NP = 10240
RPT = NP // NS

def _deg_body(src_hbm, dst_hbm, outa_hbm, outb_hbm,
              src_v, dst_v, ha_v, hb_v, sem):
    c = lax.axis_index("core")
    s = lax.axis_index("subcore")
    w = c * NS + s

    zv = jnp.zeros((16,), jnp.float32)
    ones = jnp.full((16,), 1.0, jnp.float32)

    @pl.loop(0, N // 16)
    def _(i):
        ha_v[pl.ds(i * 16, 16)] = zv
        hb_v[pl.ds(i * 16, 16)] = zv

    pltpu.async_copy(src_hbm.at[w], src_v, sem).wait()
    pltpu.async_copy(dst_hbm.at[w], dst_v, sem).wait()

    @pl.loop(0, FB)
    def _(b):
        @pl.loop(0, K // 16)
        def _(q):
            iva = src_v[b, pl.ds(q * 16, 16)]
            plsc.addupdate_scatter(ha_v, [iva], ones)
            ivb = dst_v[b, pl.ds(q * 16, 16)]
            plsc.addupdate_scatter(hb_v, [ivb], ones)

    pltpu.async_copy(ha_v, outa_hbm.at[w], sem).wait()
    pltpu.async_copy(hb_v, outb_hbm.at[w], sem).wait()


def _agg_body(h_hbm, src_hbm, dst_hbm, out_hbm,
              src_v, dst_v, msg0_v, msg1_v, acc_sh, sem, sem0, sem1):
    c = lax.axis_index("core")
    s = lax.axis_index("subcore")
    w = c * NS + s

    zv = jnp.zeros((16,), jnp.float32)

    @pl.loop(0, K)
    def _(r):
        @pl.loop(0, D // 16)
        def _(q):
            msg0_v[r, pl.ds(q * 16, 16)] = zv

    @pl.loop(0, RPT // K)
    def _(j):
        pltpu.sync_copy(msg0_v, acc_sh.at[pl.ds(s * RPT + j * K, K)])

    plsc.subcore_barrier()

    @pl.loop(0, B // BC)
    def _(ci):
        pltpu.async_copy(src_hbm.at[w, pl.ds(ci * BC, BC)], src_v, sem).wait()
        pltpu.async_copy(dst_hbm.at[w, pl.ds(ci * BC, BC)], dst_v, sem).wait()
        pltpu.async_copy(h_hbm.at[src_v.at[0]], msg0_v, sem0)

        @pl.loop(0, BC // 2 - 1)
        def _(i):
            b0 = i * 2
            pltpu.async_copy(h_hbm.at[src_v.at[b0 + 1]], msg1_v, sem1)
            pltpu.make_async_copy(h_hbm.at[src_v.at[b0]], msg0_v, sem0).wait()
            pltpu.sync_copy(msg0_v, acc_sh.at[dst_v.at[b0]], add=True)
            pltpu.async_copy(h_hbm.at[src_v.at[b0 + 2]], msg0_v, sem0)
            pltpu.make_async_copy(h_hbm.at[src_v.at[b0 + 1]], msg1_v,
                                  sem1).wait()
            pltpu.sync_copy(msg1_v, acc_sh.at[dst_v.at[b0 + 1]], add=True)

        pltpu.async_copy(h_hbm.at[src_v.at[BC - 1]], msg1_v, sem1)
        pltpu.make_async_copy(h_hbm.at[src_v.at[BC - 2]], msg0_v, sem0).wait()
        pltpu.sync_copy(msg0_v, acc_sh.at[dst_v.at[BC - 2]], add=True)
        pltpu.make_async_copy(h_hbm.at[src_v.at[BC - 1]], msg1_v, sem1).wait()
        pltpu.sync_copy(msg1_v, acc_sh.at[dst_v.at[BC - 1]], add=True)

    plsc.subcore_barrier()

    @pl.loop(0, RPT // K)
    def _(j):
        pltpu.sync_copy(acc_sh.at[pl.ds(s * RPT + j * K, K)], msg0_v)
        pltpu.sync_copy(msg0_v, out_hbm.at[c, pl.ds(s * RPT + j * K, K)])


_sc_kernels = {}


def _get_sc_kernels():
    if not _sc_kernels:
        mesh = plsc.VectorSubcoreMesh(core_axis_name="core",
                                      subcore_axis_name="subcore",
                                      num_cores=NC, num_subcores=NS)
        cp = pltpu.CompilerParams()
        if "needs_layout_passes" in pltpu.CompilerParams.__dataclass_fields__:
            cp = dataclasses.replace(cp, needs_layout_passes=False)
        _sc_kernels["deg"] = pl.kernel(
            _deg_body,
            out_type=(jax.ShapeDtypeStruct((NW, N), jnp.float32),
                      jax.ShapeDtypeStruct((NW, N), jnp.float32)),
            mesh=mesh,
            compiler_params=cp,
            scratch_types=[
                pltpu.VMEM((B, K), jnp.int32),
                pltpu.VMEM((B, K), jnp.int32),
                pltpu.VMEM((N,), jnp.float32),
                pltpu.VMEM((N,), jnp.float32),
                pltpu.SemaphoreType.DMA,
            ],
        )
        _sc_kernels["agg"] = pl.kernel(
            _agg_body,
            out_type=jax.ShapeDtypeStruct((NC, NP, D), jnp.float32),
            mesh=mesh,
            scratch_types=[
                pltpu.VMEM((BC, K), jnp.int32),
                pltpu.VMEM((BC, K), jnp.int32),
                pltpu.VMEM((K, D), jnp.float32),
                pltpu.VMEM((K, D), jnp.float32),
                pltpu.VMEM_SHARED((NP, D), jnp.float32),
                pltpu.SemaphoreType.DMA,
                pltpu.SemaphoreType.DMA,
                pltpu.SemaphoreType.DMA,
            ],
        )
    return _sc_kernels


_RB = 1000


def _norms(dega_blk, degb_blk):
    do = jnp.sum(dega_blk, axis=1)[:, None]
    di = jnp.sum(degb_blk, axis=1)[:, None]
    ns = lax.rsqrt(jnp.maximum(do, 1.0))
    nd = lax.rsqrt(jnp.maximum(di, 1.0))
    return ns, nd


def _t_first_body(dega_ref, degb_ref, x_ref, w_ref, out_ref):
    ns, _ = _norms(dega_ref[...], degb_ref[...])
    out_ref[...] = jnp.dot(x_ref[...] * ns, w_ref[...],
                           preferred_element_type=jnp.float32)


def _t_first(dega, degb, x, w):
    return pl.pallas_call(
        _t_first_body,
        grid=(N // _RB,),
        in_specs=[
            pl.BlockSpec((_RB, NW), lambda i: (i, 0)),
            pl.BlockSpec((_RB, NW), lambda i: (i, 0)),
            pl.BlockSpec((_RB, D), lambda i: (i, 0)),
            pl.BlockSpec((D, D), lambda i: (0, 0)),
        ],
        out_specs=pl.BlockSpec((_RB, D), lambda i: (i, 0)),
        out_shape=jax.ShapeDtypeStruct((N, D), jnp.float32),
    )(dega, degb, x, w)


def _t_mid_body(dega_ref, degb_ref, p_ref, b_ref, w_ref, out_ref):
    ns, nd = _norms(dega_ref[...], degb_ref[...])
    agg = p_ref[0] + p_ref[1]
    x = jnp.maximum(agg * nd + b_ref[...], 0.0)
    out_ref[...] = jnp.dot(x * ns, w_ref[...],
                           preferred_element_type=jnp.float32)


def _t_mid(dega, degb, p, bias, w):
    return pl.pallas_call(
        _t_mid_body,
        grid=(N // _RB,),
        in_specs=[
            pl.BlockSpec((_RB, NW), lambda i: (i, 0)),
            pl.BlockSpec((_RB, NW), lambda i: (i, 0)),
            pl.BlockSpec((NC, _RB, D), lambda i: (0, i, 0)),
            pl.BlockSpec((1, D), lambda i: (0, 0)),
            pl.BlockSpec((D, D), lambda i: (0, 0)),
        ],
        out_specs=pl.BlockSpec((_RB, D), lambda i: (i, 0)),
        out_shape=jax.ShapeDtypeStruct((N, D), jnp.float32),
    )(dega, degb, p, bias, w)


def _t_final_body(dega_ref, degb_ref, p_ref, b_ref, wp1_ref, bp1_ref,
                  wp2_ref, bp2_ref, out_ref):
    _, nd = _norms(dega_ref[...], degb_ref[...])
    agg = p_ref[0] + p_ref[1]
    x = jnp.maximum(agg * nd + b_ref[...], 0.0)
    m = jnp.mean(x, axis=0, keepdims=True)
    hid = jnp.maximum(jnp.dot(m, wp1_ref[...],
                              preferred_element_type=jnp.float32)
                      + bp1_ref[...], 0.0)
    out_ref[...] = jnp.dot(hid, wp2_ref[...],
                           preferred_element_type=jnp.float32) + bp2_ref[...]


def _t_final(dega, degb, p, bias, wp1, bp1, wp2, bp2):
    n_class = wp2.shape[1]
    mid = wp1.shape[1]
    return pl.pallas_call(
        _t_final_body,
        grid=(1,),
        in_specs=[
            pl.BlockSpec((N, NW), lambda i: (0, 0)),
            pl.BlockSpec((N, NW), lambda i: (0, 0)),
            pl.BlockSpec((NC, N, D), lambda i: (0, 0, 0)),
            pl.BlockSpec((1, D), lambda i: (0, 0)),
            pl.BlockSpec((D, mid), lambda i: (0, 0)),
            pl.BlockSpec((1, mid), lambda i: (0, 0)),
            pl.BlockSpec((mid, n_class), lambda i: (0, 0)),
            pl.BlockSpec((1, n_class), lambda i: (0, 0)),
        ],
        out_specs=pl.BlockSpec((1, n_class), lambda i: (0, 0)),
        out_shape=jax.ShapeDtypeStruct((1, n_class), jnp.float32),
    )(dega, degb, p, bias, wp1, bp1, wp2, bp2)


def kernel(feat, edge_index, W1, b1, W2, b2, W3, b3, W4, b4, W5, b5, W6, b6,
           Wp1, bp1, Wp2, bp2):
    src_pad = jnp.broadcast_to(jnp.arange(PAD, dtype=jnp.int32), (NW, PAD))
    dst_pad = jnp.broadcast_to(N + jnp.arange(PAD, dtype=jnp.int32), (NW, PAD))
    src = jnp.concatenate([edge_index[0].reshape(NW, EPW), src_pad],
                          axis=1).reshape(NW, B, K)
    dst = jnp.concatenate([edge_index[1].reshape(NW, EPW), dst_pad],
                          axis=1).reshape(NW, B, K)
    sck = _get_sc_kernels()
    dega, degb = sck["deg"](src, dst)
    dega = dega.T
    degb = degb.T

    h = _t_first(dega, degb, feat, W1)
    mids = [(b1, W2), (b2, W3), (b3, W4), (b4, W5), (b5, W6)]
    for bias, w in mids:
        p = sck["agg"](h, src, dst)
        h = _t_mid(dega, degb, p, bias.reshape(1, D), w)
    p = sck["agg"](h, src, dst)

    return _t_final(dega, degb, p, b6.reshape(1, D), Wp1, bp1.reshape(1, -1),
                    Wp2, bp2.reshape(1, -1))
